# Initial kernel scaffold; baseline (speedup 1.0000x reference)
import jax
import jax.numpy as jnp
from jax.experimental import pallas as pl


def kernel(x, weight):
    raise NotImplementedError("write your pallas kernel here")


# speedup vs baseline: 1.1032x; 1.1032x over previous
import functools

import jax
import jax.numpy as jnp
from jax import lax
from jax.experimental import pallas as pl
from jax.experimental.pallas import tpu as pltpu
from jax.experimental.pallas import tpu_sc as plsc

NUM_CORES = 2
NUM_SUBCORES = 16
NUM_WORKERS = NUM_CORES * NUM_SUBCORES
ROWS_PER_GROUP = 128
GROUPS_PER_STEP = 8


@functools.cache
def _make_kernel(n_rows: int, dim: int):
    rows_per_w = n_rows // NUM_WORKERS
    groups = rows_per_w // ROWS_PER_GROUP
    steps = groups // GROUPS_PER_STEP
    step_rows = GROUPS_PER_STEP * ROWS_PER_GROUP

    mesh = plsc.VectorSubcoreMesh(
        core_axis_name="c", subcore_axis_name="s",
        num_cores=NUM_CORES, num_subcores=NUM_SUBCORES)

    @functools.partial(
        pl.kernel,
        out_type=jax.ShapeDtypeStruct((n_rows, dim), jnp.float32),
        mesh=mesh,
        scratch_types=[
            pltpu.VMEM((groups, ROWS_PER_GROUP), jnp.int32),
            pltpu.VMEM((step_rows, dim), jnp.float32),
            pltpu.SemaphoreType.DMA,
        ],
        compiler_params=pltpu.CompilerParams(use_tc_tiling_on_sc=False),
    )
    def emb(idx_hbm, table_hbm, out_hbm, idx_v, rows_v, sem):
        wid = lax.axis_index("s") * NUM_CORES + lax.axis_index("c")
        base = wid * rows_per_w
        pltpu.sync_copy(idx_hbm.at[wid], idx_v)

        def step(s, carry):
            copies = []
            for g in range(GROUPS_PER_STEP):
                copies.append(pltpu.async_copy(
                    table_hbm.at[idx_v.at[s * GROUPS_PER_STEP + g]],
                    rows_v.at[pl.ds(g * ROWS_PER_GROUP, ROWS_PER_GROUP)],
                    sem))
            for c in copies:
                c.wait()
            pltpu.sync_copy(
                rows_v,
                out_hbm.at[pl.ds(base + s * step_rows, step_rows)])
            return carry

        lax.fori_loop(0, steps, step, 0)

    return emb


def kernel(x, weight):
    b, l = x.shape
    _, dim = weight.shape
    n_rows = b * l
    idx = x.reshape(
        NUM_WORKERS, n_rows // NUM_WORKERS // ROWS_PER_GROUP, ROWS_PER_GROUP
    ).astype(jnp.int32)
    out = _make_kernel(n_rows, dim)(idx, weight)
    return out.reshape(b, l, dim)


# speedup vs baseline: 1.1101x; 1.0063x over previous
import functools

import jax
import jax.numpy as jnp
from jax import lax
from jax.experimental import pallas as pl
from jax.experimental.pallas import tpu as pltpu
from jax.experimental.pallas import tpu_sc as plsc

NUM_CORES = 2
NUM_SUBCORES = 16
NUM_WORKERS = NUM_CORES * NUM_SUBCORES
ROWS_PER_GROUP = 128
GROUPS_PER_STEP = 10


@functools.cache
def _make_kernel(n_rows: int, dim: int):
    rows_per_w = n_rows // NUM_WORKERS
    groups = rows_per_w // ROWS_PER_GROUP
    steps = groups // GROUPS_PER_STEP
    assert steps % 2 == 0 and steps >= 4
    step_rows = GROUPS_PER_STEP * ROWS_PER_GROUP

    mesh = plsc.VectorSubcoreMesh(
        core_axis_name="c", subcore_axis_name="s",
        num_cores=NUM_CORES, num_subcores=NUM_SUBCORES)

    @functools.partial(
        pl.kernel,
        out_type=jax.ShapeDtypeStruct((n_rows, dim), jnp.float32),
        mesh=mesh,
        scratch_types=[
            pltpu.VMEM((groups, ROWS_PER_GROUP), jnp.int32),
            pltpu.VMEM((2, step_rows, dim), jnp.float32),
            pltpu.SemaphoreType.DMA,
            pltpu.SemaphoreType.DMA,
            pltpu.SemaphoreType.DMA,
            pltpu.SemaphoreType.DMA,
        ],
        compiler_params=pltpu.CompilerParams(use_tc_tiling_on_sc=False),
    )
    def emb(idx_hbm, table_hbm, out_hbm, idx_v, rows_v, gs0, gs1, ws0, ws1):
        wid = lax.axis_index("s") * NUM_CORES + lax.axis_index("c")
        base = wid * rows_per_w
        pltpu.sync_copy(idx_hbm.at[wid], idx_v)

        def fire(step, b, gsem):
            for g in range(GROUPS_PER_STEP):
                pltpu.async_copy(
                    table_hbm.at[idx_v.at[step * GROUPS_PER_STEP + g]],
                    rows_v.at[b].at[pl.ds(g * ROWS_PER_GROUP, ROWS_PER_GROUP)],
                    gsem)

        def wait_gather(b, gsem):
            pltpu.make_async_copy(
                out_hbm.at[pl.ds(0, step_rows)], rows_v.at[b], gsem).wait()

        def start_wb(step, b, wsem):
            pltpu.async_copy(
                rows_v.at[b],
                out_hbm.at[pl.ds(base + step * step_rows, step_rows)], wsem)

        def wait_wb(b, wsem):
            pltpu.make_async_copy(
                rows_v.at[b], out_hbm.at[pl.ds(base, step_rows)], wsem).wait()

        fire(0, 0, gs0)
        fire(1, 1, gs1)

        def body(j, carry):
            s0 = 2 * j
            wait_gather(0, gs0)
            start_wb(s0, 0, ws0)
            wait_gather(1, gs1)
            start_wb(s0 + 1, 1, ws1)
            wait_wb(0, ws0)
            fire(s0 + 2, 0, gs0)
            wait_wb(1, ws1)
            fire(s0 + 3, 1, gs1)
            return carry

        lax.fori_loop(0, steps // 2 - 1, body, 0)

        wait_gather(0, gs0)
        start_wb(steps - 2, 0, ws0)
        wait_gather(1, gs1)
        start_wb(steps - 1, 1, ws1)
        wait_wb(0, ws0)
        wait_wb(1, ws1)

    return emb


def kernel(x, weight):
    b, l = x.shape
    _, dim = weight.shape
    n_rows = b * l
    idx = x.reshape(
        NUM_WORKERS, n_rows // NUM_WORKERS // ROWS_PER_GROUP, ROWS_PER_GROUP
    ).astype(jnp.int32)
    out = _make_kernel(n_rows, dim)(idx, weight)
    return out.reshape(b, l, dim)


# speedup vs baseline: 1.1103x; 1.0002x over previous
import functools

import jax
import jax.numpy as jnp
from jax import lax
from jax.experimental import pallas as pl
from jax.experimental.pallas import tpu as pltpu
from jax.experimental.pallas import tpu_sc as plsc

NUM_CORES = 2
NUM_SUBCORES = 16
NUM_WORKERS = NUM_CORES * NUM_SUBCORES
ROWS_PER_GROUP = 1280
GROUPS_PER_STEP = 1


@functools.cache
def _make_kernel(n_rows: int, dim: int):
    rows_per_w = n_rows // NUM_WORKERS
    groups = rows_per_w // ROWS_PER_GROUP
    steps = groups // GROUPS_PER_STEP
    assert steps % 2 == 0 and steps >= 4
    step_rows = GROUPS_PER_STEP * ROWS_PER_GROUP

    mesh = plsc.VectorSubcoreMesh(
        core_axis_name="c", subcore_axis_name="s",
        num_cores=NUM_CORES, num_subcores=NUM_SUBCORES)

    @functools.partial(
        pl.kernel,
        out_type=jax.ShapeDtypeStruct((n_rows, dim), jnp.float32),
        mesh=mesh,
        scratch_types=[
            pltpu.VMEM((groups, ROWS_PER_GROUP), jnp.int32),
            pltpu.VMEM((2, step_rows, dim), jnp.float32),
            pltpu.SemaphoreType.DMA,
            pltpu.SemaphoreType.DMA,
            pltpu.SemaphoreType.DMA,
            pltpu.SemaphoreType.DMA,
        ],
        compiler_params=pltpu.CompilerParams(use_tc_tiling_on_sc=False),
    )
    def emb(idx_hbm, table_hbm, out_hbm, idx_v, rows_v, gs0, gs1, ws0, ws1):
        wid = lax.axis_index("s") * NUM_CORES + lax.axis_index("c")
        base = wid * rows_per_w
        pltpu.sync_copy(idx_hbm.at[wid], idx_v)

        def fire(step, b, gsem):
            for g in range(GROUPS_PER_STEP):
                pltpu.async_copy(
                    table_hbm.at[idx_v.at[step * GROUPS_PER_STEP + g]],
                    rows_v.at[b].at[pl.ds(g * ROWS_PER_GROUP, ROWS_PER_GROUP)],
                    gsem)

        def wait_gather(b, gsem):
            pltpu.make_async_copy(
                out_hbm.at[pl.ds(0, step_rows)], rows_v.at[b], gsem).wait()

        def start_wb(step, b, wsem):
            pltpu.async_copy(
                rows_v.at[b],
                out_hbm.at[pl.ds(base + step * step_rows, step_rows)], wsem)

        def wait_wb(b, wsem):
            pltpu.make_async_copy(
                rows_v.at[b], out_hbm.at[pl.ds(base, step_rows)], wsem).wait()

        fire(0, 0, gs0)
        fire(1, 1, gs1)

        def body(j, carry):
            s0 = 2 * j
            wait_gather(0, gs0)
            start_wb(s0, 0, ws0)
            wait_gather(1, gs1)
            start_wb(s0 + 1, 1, ws1)
            wait_wb(0, ws0)
            fire(s0 + 2, 0, gs0)
            wait_wb(1, ws1)
            fire(s0 + 3, 1, gs1)
            return carry

        lax.fori_loop(0, steps // 2 - 1, body, 0)

        wait_gather(0, gs0)
        start_wb(steps - 2, 0, ws0)
        wait_gather(1, gs1)
        start_wb(steps - 1, 1, ws1)
        wait_wb(0, ws0)
        wait_wb(1, ws1)

    return emb


def kernel(x, weight):
    b, l = x.shape
    _, dim = weight.shape
    n_rows = b * l
    idx = x.reshape(
        NUM_WORKERS, n_rows // NUM_WORKERS // ROWS_PER_GROUP, ROWS_PER_GROUP
    ).astype(jnp.int32)
    out = _make_kernel(n_rows, dim)(idx, weight)
    return out.reshape(b, l, dim)


# speedup vs baseline: 1.7896x; 1.6118x over previous
import functools

import jax
import jax.numpy as jnp
from jax import lax
from jax.experimental import pallas as pl
from jax.experimental.pallas import tpu as pltpu
from jax.experimental.pallas import tpu_sc as plsc

NUM_CORES = 2
NUM_SUBCORES = 16
NUM_WORKERS = NUM_CORES * NUM_SUBCORES


@functools.cache
def _make_kernel(b: int, l: int, dim: int):
    b_per_w = b // NUM_WORKERS
    assert l % 2 == 0 and b_per_w % 8 == 0

    mesh = plsc.VectorSubcoreMesh(
        core_axis_name="c", subcore_axis_name="s",
        num_cores=NUM_CORES, num_subcores=NUM_SUBCORES)

    @functools.partial(
        pl.kernel,
        out_type=jax.ShapeDtypeStruct((b, l, dim), jnp.float32),
        mesh=mesh,
        scratch_types=[
            pltpu.VMEM((l, b_per_w), jnp.int32),
            pltpu.VMEM((2, b_per_w, dim), jnp.float32),
            pltpu.SemaphoreType.DMA,
            pltpu.SemaphoreType.DMA,
            pltpu.SemaphoreType.DMA,
            pltpu.SemaphoreType.DMA,
        ],
        compiler_params=pltpu.CompilerParams(use_tc_tiling_on_sc=False),
    )
    def emb(xt_hbm, table_hbm, out_hbm, xv, rows, gs0, gs1, ws0, ws1):
        wid = lax.axis_index("s") * NUM_CORES + lax.axis_index("c")
        b0 = wid * b_per_w
        pltpu.sync_copy(xt_hbm.at[:, pl.ds(b0, b_per_w)], xv)

        gsems = (gs0, gs1)
        wsems = (ws0, ws1)

        def fire(pos, buf):
            pltpu.async_copy(
                table_hbm.at[xv.at[pos]], rows.at[buf], gsems[buf])

        def wait_gather(buf):
            pltpu.make_async_copy(
                table_hbm.at[xv.at[0]], rows.at[buf], gsems[buf]).wait()

        def start_wb(pos, buf):
            pltpu.async_copy(
                rows.at[buf], out_hbm.at[pl.ds(b0, b_per_w), pos],
                wsems[buf])

        def wait_wb(buf):
            pltpu.make_async_copy(
                rows.at[buf], out_hbm.at[pl.ds(b0, b_per_w), 0],
                wsems[buf]).wait()

        fire(0, 0)

        def body(j, carry):
            p0 = 2 * j
            wait_gather(0)
            start_wb(p0, 0)

            @pl.when(j > 0)
            def _():
                wait_wb(1)

            fire(p0 + 1, 1)
            wait_gather(1)
            start_wb(p0 + 1, 1)
            wait_wb(0)

            @pl.when(j < l // 2 - 1)
            def _():
                fire(p0 + 2, 0)

            return carry

        lax.fori_loop(0, l // 2, body, 0)
        wait_wb(1)

    return emb


def kernel(x, weight):
    b, l = x.shape
    _, dim = weight.shape
    xt = x.T.astype(jnp.int32)
    return _make_kernel(b, l, dim)(xt, weight)
